# baseline (speedup 1.0000x reference)
import functools

import jax
import jax.numpy as jnp
from jax import lax
from jax.experimental import pallas as pl
from jax.experimental.pallas import tpu as pltpu
from jax.experimental.pallas import tpu_sc as plsc

B, L, D = 16384, 7, 64
R = B * L
NW = 32
RPW = R // NW
CHUNK = 896
NCHUNK = RPW // CHUNK
GSZ = 128
NSUB = CHUNK // GSZ
NLANE = 16
NVPR = D // NLANE


def _build_sc_kernel():
    mesh = plsc.VectorSubcoreMesh(core_axis_name="c", subcore_axis_name="s")

    @functools.partial(
        pl.kernel,
        mesh=mesh,
        out_type=jax.ShapeDtypeStruct((R, D), jnp.float32),
        scratch_types=[
            pltpu.VMEM((1, NSUB * NCHUNK, GSZ), jnp.int32),
            pltpu.VMEM((CHUNK, D), jnp.float32),
            pltpu.VMEM((L, D), jnp.float32),
            pltpu.SemaphoreType.DMA,
        ],
        compiler_params=pltpu.CompilerParams(use_tc_tiling_on_sc=False),
    )
    def sc_embed(x_hbm, tok_hbm, pos_hbm, out_hbm, idx_v, rows_v, pos_v, sem):
        wid = lax.axis_index("s") * 2 + lax.axis_index("c")
        pltpu.sync_copy(pos_hbm, pos_v)
        pltpu.sync_copy(x_hbm.at[pl.ds(wid, 1)], idx_v)
        pvals = [[pos_v[p, pl.ds(c * NLANE, NLANE)] for c in range(NVPR)]
                 for p in range(L)]

        for kck in range(NCHUNK):
            copies = []
            for j in range(NSUB):
                copies.append(pltpu.async_copy(
                    tok_hbm.at[idx_v.at[0, kck * NSUB + j]],
                    rows_v.at[pl.ds(j * GSZ, GSZ)],
                    sem))
            for cp in copies:
                cp.wait()

            def body(g, carry):
                for p in range(L):
                    r = g * L + p
                    for c in range(NVPR):
                        plsc.addupdate(rows_v.at[r, pl.ds(c * NLANE, NLANE)],
                                       pvals[p][c])
                return carry

            lax.fori_loop(0, CHUNK // L, body, 0)
            pltpu.sync_copy(rows_v,
                            out_hbm.at[pl.ds(wid * RPW + kck * CHUNK, CHUNK)])

    return sc_embed


_sc_embed = _build_sc_kernel()


def kernel(x, token_table, pos_table):
    x3d = x.astype(jnp.int32).reshape(NW, NSUB * NCHUNK, GSZ)
    out = _sc_embed(x3d, token_table, pos_table)
    return out.reshape(B, L, D)


# speedup vs baseline: 1.0004x; 1.0004x over previous
import functools

import jax
import jax.numpy as jnp
from jax import lax
from jax.experimental import pallas as pl
from jax.experimental.pallas import tpu as pltpu
from jax.experimental.pallas import tpu_sc as plsc

B, L, D = 16384, 7, 64
R = B * L
NW = 32
JPW = R // NW
CHUNK = 512
NCHUNK = JPW // CHUNK
GSZ = 128
NSUB = CHUNK // GSZ
NLANE = 16
NVPR = D // NLANE


def _build_sc_kernel():
    mesh = plsc.VectorSubcoreMesh(core_axis_name="c", subcore_axis_name="s")

    @functools.partial(
        pl.kernel,
        mesh=mesh,
        out_type=jax.ShapeDtypeStruct((B, L, D), jnp.float32),
        scratch_types=[
            pltpu.VMEM((NSUB, GSZ), jnp.int32),
            pltpu.VMEM((CHUNK, D), jnp.float32),
            pltpu.VMEM((L, D), jnp.float32),
            pltpu.SemaphoreType.DMA,
        ],
        compiler_params=pltpu.CompilerParams(use_tc_tiling_on_sc=False),
    )
    def sc_embed(xt_hbm, tok_hbm, pos_hbm, out_hbm, idx_v, rows_v, pos_v, sem):
        wid = lax.axis_index("s") * 2 + lax.axis_index("c")
        pltpu.sync_copy(pos_hbm, pos_v)

        for kck in range(NCHUNK):
            m = wid * NCHUNK + kck
            l = m // (B // CHUNK)
            b0 = (m % (B // CHUNK)) * CHUNK
            pltpu.sync_copy(xt_hbm.at[pl.ds(NSUB * m, NSUB)], idx_v)
            copies = []
            for j in range(NSUB):
                copies.append(pltpu.async_copy(
                    tok_hbm.at[idx_v.at[j]],
                    rows_v.at[pl.ds(j * GSZ, GSZ)],
                    sem))
            for cp in copies:
                cp.wait()

            pv = [pos_v[l, pl.ds(c * NLANE, NLANE)] for c in range(NVPR)]

            def body(g, carry):
                for u in range(8):
                    r = g * 8 + u
                    for c in range(NVPR):
                        plsc.addupdate(rows_v.at[r, pl.ds(c * NLANE, NLANE)],
                                       pv[c])
                return carry

            lax.fori_loop(0, CHUNK // 8, body, 0)
            pltpu.sync_copy(rows_v, out_hbm.at[pl.ds(b0, CHUNK), l])

    return sc_embed


_sc_embed = _build_sc_kernel()


def kernel(x, token_table, pos_table):
    xt2 = x.astype(jnp.int32).T.reshape(R // GSZ, GSZ)
    return _sc_embed(xt2, token_table, pos_table)
